# Initial kernel scaffold; baseline (speedup 1.0000x reference)
import jax
import jax.numpy as jnp
from jax.experimental import pallas as pl


def kernel(x, emb, W, b):
    raise NotImplementedError("write your pallas kernel here")


# speedup vs baseline: 5.8891x; 5.8891x over previous
import functools

import jax
import jax.numpy as jnp
from jax import lax
from jax.experimental import pallas as pl
from jax.experimental.pallas import tpu as pltpu
from jax.experimental.pallas import tpu_sc as plsc

_NC = 2
_NS = 16
_NW = _NC * _NS
_LANES = 16


@functools.lru_cache(maxsize=None)
def _build(n: int, cb: int):
    assert n % (_NW * cb) == 0
    pw = n // _NW
    ng = pw // cb
    mesh = plsc.VectorSubcoreMesh(core_axis_name="c", subcore_axis_name="s",
                                  num_cores=_NC, num_subcores=_NS)

    @functools.partial(
        pl.kernel,
        out_type=jax.ShapeDtypeStruct((3 * n,), jnp.float32),
        mesh=mesh,
        scratch_types=[
            pltpu.VMEM((48,), jnp.float32),
            pltpu.VMEM((32,), jnp.float32),
            pltpu.VMEM((cb,), jnp.int32),
            pltpu.VMEM((cb,), jnp.int32),
            pltpu.VMEM((3 * cb,), jnp.float32),
            pltpu.VMEM((3 * cb,), jnp.float32),
            pltpu.SemaphoreType.DMA,
            pltpu.SemaphoreType.DMA,
            pltpu.SemaphoreType.DMA,
            pltpu.SemaphoreType.DMA,
        ],
        compiler_params=pltpu.CompilerParams(needs_layout_passes=False),
    )
    def emb_linear_sc(x_hbm, p_hbm, out_hbm,
                      p_v, ft_v, xb0, xb1, ob0, ob1, si0, si1, so0, so1):
        wid = lax.axis_index("s") * _NC + lax.axis_index("c")
        base = wid * pw
        lane = jnp.arange(_LANES, dtype=jnp.int32)
        lane3 = lane * 3

        pltpu.sync_copy(p_hbm, p_v)

        def ft_half(e0):
            e = lane + e0
            k = jnp.minimum(e // 3, 7)
            c = e % 3
            acc = plsc.load_gather(p_v, [44 + c])
            for d in range(4):
                acc = acc + (plsc.load_gather(p_v, [k * 4 + d])
                             * plsc.load_gather(p_v, [32 + c * 4 + d]))
            return acc

        ft_v[pl.ds(0, _LANES)] = ft_half(0)
        ft_v[pl.ds(_LANES, _LANES)] = ft_half(_LANES)

        xbufs, obufs = (xb0, xb1), (ob0, ob1)
        sins, souts = (si0, si1), (so0, so1)
        in_d, out_d = {}, {}

        def start_in(g):
            d = pltpu.make_async_copy(
                x_hbm.at[pl.ds(base + g * cb, cb)], xbufs[g % 2], sins[g % 2])
            d.start()
            in_d[g] = d

        def start_out(g):
            d = pltpu.make_async_copy(
                obufs[g % 2],
                out_hbm.at[pl.ds((base + g * cb) * 3, 3 * cb)], souts[g % 2])
            d.start()
            out_d[g] = d

        def compute(g):
            xb, ob = xbufs[g % 2], obufs[g % 2]

            @plsc.parallel_loop(0, cb // _LANES, 1, unroll=4)
            def _(i):
                xv = xb[pl.ds(i * _LANES, _LANES)]
                key = xv * 3
                pos = i * (3 * _LANES) + lane3
                for c in range(3):
                    val = plsc.load_gather(ft_v, [key + c])
                    plsc.store_scatter(ob, [pos + c], val)

        start_in(0)
        for g in range(ng):
            if g + 1 < ng:
                start_in(g + 1)
            in_d[g].wait()
            if g >= 2:
                out_d[g - 2].wait()
            compute(g)
            start_out(g)
        out_d[ng - 2].wait()
        out_d[ng - 1].wait()

    return emb_linear_sc


def kernel(x, emb, W, b):
    bsz, seq = x.shape
    n = bsz * seq
    params = jnp.concatenate(
        [emb.reshape(-1), W.reshape(-1), b.reshape(-1),
         jnp.zeros((1,), jnp.float32)])
    out_flat = _build(n, 6400)(x.reshape(-1), params)
    return out_flat.reshape(bsz, seq, 3)


# speedup vs baseline: 283.9083x; 48.2095x over previous
import functools

import jax
import jax.numpy as jnp
from jax import lax
from jax.experimental import pallas as pl
from jax.experimental.pallas import tpu as pltpu
from jax.experimental.pallas import tpu_sc as plsc

_NC = 2
_NS = 16
_NW = _NC * _NS
_LANES = 16


@functools.lru_cache(maxsize=None)
def _build(n: int, cb: int):
    assert n % (_NW * cb) == 0
    pw = n // _NW
    ng = pw // cb
    mesh = plsc.VectorSubcoreMesh(core_axis_name="c", subcore_axis_name="s",
                                  num_cores=_NC, num_subcores=_NS)

    @functools.partial(
        pl.kernel,
        out_type=jax.ShapeDtypeStruct((3 * n,), jnp.float32),
        mesh=mesh,
        scratch_types=[
            pltpu.VMEM((48,), jnp.float32),
            pltpu.VMEM((32,), jnp.float32),
            pltpu.VMEM((384,), jnp.float32),
            pltpu.VMEM((cb,), jnp.int32),
            pltpu.VMEM((cb,), jnp.int32),
            pltpu.VMEM((3 * cb,), jnp.float32),
            pltpu.VMEM((3 * cb,), jnp.float32),
            pltpu.SemaphoreType.DMA,
            pltpu.SemaphoreType.DMA,
            pltpu.SemaphoreType.DMA,
            pltpu.SemaphoreType.DMA,
        ],
        compiler_params=pltpu.CompilerParams(needs_layout_passes=False),
    )
    def emb_linear_sc(x_hbm, p_hbm, out_hbm, p_v, ft_v, tbl,
                      xb0, xb1, ob0, ob1, si0, si1, so0, so1):
        wid = lax.axis_index("s") * _NC + lax.axis_index("c")
        base = wid * pw
        lane = jnp.arange(_LANES, dtype=jnp.int32)

        pltpu.sync_copy(p_hbm, p_v)

        def ft_half(e0):
            e = lane + e0
            k = jnp.minimum(e // 3, 7)
            c = e % 3
            acc = plsc.load_gather(p_v, [44 + c])
            for d in range(4):
                acc = acc + (plsc.load_gather(p_v, [k * 4 + d])
                             * plsc.load_gather(p_v, [32 + c * 4 + d]))
            return acc

        ftv0 = ft_half(0)
        ftv1 = ft_half(_LANES)
        ft_v[pl.ds(0, _LANES)] = ftv0
        ft_v[pl.ds(_LANES, _LANES)] = ftv1

        for r in range(_LANES):
            plsc.store_scatter(tbl, [lane * _LANES + r], ftv0)
            plsc.store_scatter(tbl, [(_LANES + lane) * _LANES + r], ftv1,
                               mask=lane < 8)

        xbufs, obufs = (xb0, xb1), (ob0, ob1)
        sins, souts = (si0, si1), (so0, so1)
        in_d, out_d = {}, {}

        def start_in(g):
            d = pltpu.make_async_copy(
                x_hbm.at[pl.ds(base + g * cb, cb)], xbufs[g % 2], sins[g % 2])
            d.start()
            in_d[g] = d

        def start_out(g):
            ds = []
            for c in range(3):
                d = pltpu.make_async_copy(
                    obufs[g % 2].at[pl.ds(c * cb, cb)],
                    out_hbm.at[pl.ds(c * n + base + g * cb, cb)],
                    souts[g % 2])
                d.start()
                ds.append(d)
            out_d[g] = ds

        def compute(g):
            xb, ob = xbufs[g % 2], obufs[g % 2]

            @plsc.parallel_loop(0, cb // _LANES, 1, unroll=4)
            def _(i):
                xv = xb[pl.ds(i * _LANES, _LANES)]
                idx = xv * (3 * _LANES) + lane
                for c in range(3):
                    ob[pl.ds(c * cb + i * _LANES, _LANES)] = (
                        plsc.load_gather(tbl, [idx + c * _LANES]))

        start_in(0)
        for g in range(ng):
            if g + 1 < ng:
                start_in(g + 1)
            in_d[g].wait()
            if g >= 2:
                for d in out_d[g - 2]:
                    d.wait()
            compute(g)
            start_out(g)
        for g in (ng - 2, ng - 1):
            for d in out_d[g]:
                d.wait()

    return emb_linear_sc


def kernel(x, emb, W, b):
    bsz, seq = x.shape
    n = bsz * seq
    it, im = bsz // 128, 128
    jt, js = seq // 8, 8
    xp = (x.reshape(it, im, jt, js)
           .transpose(2, 0, 3, 1)
           .reshape(-1))
    params = jnp.concatenate(
        [emb.reshape(-1), W.reshape(-1), b.reshape(-1),
         jnp.zeros((1,), jnp.float32)])
    op = _build(n, 6400)(xp, params)
    out = (op.reshape(3, jt, it, js, im)
             .transpose(2, 4, 1, 3, 0)
             .reshape(bsz, seq, 3))
    return out


# speedup vs baseline: 293.0942x; 1.0324x over previous
import functools

import jax
import jax.numpy as jnp
from jax import lax
from jax.experimental import pallas as pl
from jax.experimental.pallas import tpu as pltpu
from jax.experimental.pallas import tpu_sc as plsc

_NC = 2
_NS = 16
_NW = _NC * _NS
_LANES = 16


@functools.lru_cache(maxsize=None)
def _build(n: int, cb: int):
    assert n % (_NW * cb) == 0
    pw = n // _NW
    ng = pw // cb
    mesh = plsc.VectorSubcoreMesh(core_axis_name="c", subcore_axis_name="s",
                                  num_cores=_NC, num_subcores=_NS)

    @functools.partial(
        pl.kernel,
        out_type=jax.ShapeDtypeStruct((3 * n,), jnp.float32),
        mesh=mesh,
        scratch_types=[
            pltpu.VMEM((48,), jnp.float32),
            pltpu.VMEM((32,), jnp.float32),
            pltpu.VMEM((384,), jnp.float32),
            pltpu.VMEM((cb,), jnp.int32),
            pltpu.VMEM((cb,), jnp.int32),
            pltpu.VMEM((3 * cb,), jnp.float32),
            pltpu.VMEM((3 * cb,), jnp.float32),
            pltpu.SemaphoreType.DMA,
            pltpu.SemaphoreType.DMA,
            pltpu.SemaphoreType.DMA,
            pltpu.SemaphoreType.DMA,
        ],
        compiler_params=pltpu.CompilerParams(needs_layout_passes=False),
    )
    def emb_linear_sc(x_hbm, p_hbm, out_hbm, p_v, ft_v, tbl,
                      xb0, xb1, ob0, ob1, si0, si1, so0, so1):
        wid = lax.axis_index("s") * _NC + lax.axis_index("c")
        base = wid * pw
        lane = jnp.arange(_LANES, dtype=jnp.int32)

        pltpu.sync_copy(p_hbm, p_v)

        def ft_half(e0):
            e = lane + e0
            k = jnp.minimum(e // 3, 7)
            c = e % 3
            acc = plsc.load_gather(p_v, [44 + c])
            for d in range(4):
                acc = acc + (plsc.load_gather(p_v, [k * 4 + d])
                             * plsc.load_gather(p_v, [32 + c * 4 + d]))
            return acc

        ftv0 = ft_half(0)
        ftv1 = ft_half(_LANES)
        ft_v[pl.ds(0, _LANES)] = ftv0
        ft_v[pl.ds(_LANES, _LANES)] = ftv1

        for r in range(_LANES):
            plsc.store_scatter(tbl, [lane * _LANES + r], ftv0)
            plsc.store_scatter(tbl, [(_LANES + lane) * _LANES + r], ftv1,
                               mask=lane < 8)

        xbufs, obufs = (xb0, xb1), (ob0, ob1)
        sins, souts = (si0, si1), (so0, so1)
        in_d, out_d = {}, {}

        def start_in(g):
            d = pltpu.make_async_copy(
                x_hbm.at[pl.ds(base + g * cb, cb)], xbufs[g % 2], sins[g % 2])
            d.start()
            in_d[g] = d

        def start_out(g):
            ds = []
            for c in range(3):
                d = pltpu.make_async_copy(
                    obufs[g % 2].at[pl.ds(c * cb, cb)],
                    out_hbm.at[pl.ds(c * n + base + g * cb, cb)],
                    souts[g % 2])
                d.start()
                ds.append(d)
            out_d[g] = ds

        def compute(g):
            xb, ob = xbufs[g % 2], obufs[g % 2]

            @plsc.parallel_loop(0, cb // _LANES, 1, unroll=8)
            def _(i):
                xv = xb[pl.ds(i * _LANES, _LANES)]
                idx = xv * (3 * _LANES) + lane
                for c in range(3):
                    ob[pl.ds(c * cb + i * _LANES, _LANES)] = (
                        plsc.load_gather(tbl, [idx + c * _LANES]))

        start_in(0)
        for g in range(ng):
            if g + 1 < ng:
                start_in(g + 1)
            in_d[g].wait()
            if g >= 2:
                for d in out_d[g - 2]:
                    d.wait()
            compute(g)
            start_out(g)
        for g in (ng - 2, ng - 1):
            for d in out_d[g]:
                d.wait()

    return emb_linear_sc


def kernel(x, emb, W, b):
    bsz, seq = x.shape
    n = bsz * seq
    it, im = bsz // 128, 128
    jt, js = seq // 8, 8
    xp = (x.reshape(it, im, jt, js)
           .transpose(2, 0, 3, 1)
           .reshape(-1))
    params = jnp.concatenate(
        [emb.reshape(-1), W.reshape(-1), b.reshape(-1),
         jnp.zeros((1,), jnp.float32)])
    op = _build(n, 12800)(xp, params)
    out = (op.reshape(3, jt, it, js, im)
             .transpose(2, 4, 1, 3, 0)
             .reshape(bsz, seq, 3))
    return out


# speedup vs baseline: 293.6606x; 1.0019x over previous
import functools

import jax
import jax.numpy as jnp
from jax import lax
from jax.experimental import pallas as pl
from jax.experimental.pallas import tpu as pltpu
from jax.experimental.pallas import tpu_sc as plsc

_NC = 2
_NS = 16
_NW = _NC * _NS
_LANES = 16


@functools.lru_cache(maxsize=None)
def _build(n: int, cb: int):
    assert n % (_NW * cb) == 0
    pw = n // _NW
    ng = pw // cb
    mesh = plsc.VectorSubcoreMesh(core_axis_name="c", subcore_axis_name="s",
                                  num_cores=_NC, num_subcores=_NS)

    @functools.partial(
        pl.kernel,
        out_type=jax.ShapeDtypeStruct((3 * n,), jnp.float32),
        mesh=mesh,
        scratch_types=[
            pltpu.VMEM((48,), jnp.float32),
            pltpu.VMEM((32,), jnp.float32),
            pltpu.VMEM((384,), jnp.float32),
            pltpu.VMEM((cb,), jnp.int32),
            pltpu.VMEM((cb,), jnp.int32),
            pltpu.VMEM((3 * cb,), jnp.float32),
            pltpu.VMEM((3 * cb,), jnp.float32),
            pltpu.SemaphoreType.DMA,
            pltpu.SemaphoreType.DMA,
            pltpu.SemaphoreType.DMA,
            pltpu.SemaphoreType.DMA,
        ],
        compiler_params=pltpu.CompilerParams(
            needs_layout_passes=False,
            disable_bounds_checks=True,
            disable_semaphore_checks=True),
    )
    def emb_linear_sc(x_hbm, p_hbm, out_hbm, p_v, ft_v, tbl,
                      xb0, xb1, ob0, ob1, si0, si1, so0, so1):
        wid = lax.axis_index("s") * _NC + lax.axis_index("c")
        base = wid * pw
        lane = jnp.arange(_LANES, dtype=jnp.int32)

        pltpu.sync_copy(p_hbm, p_v)

        def ft_half(e0):
            e = lane + e0
            k = jnp.minimum(e // 3, 7)
            c = e % 3
            acc = plsc.load_gather(p_v, [44 + c])
            for d in range(4):
                acc = acc + (plsc.load_gather(p_v, [k * 4 + d])
                             * plsc.load_gather(p_v, [32 + c * 4 + d]))
            return acc

        ftv0 = ft_half(0)
        ftv1 = ft_half(_LANES)
        ft_v[pl.ds(0, _LANES)] = ftv0
        ft_v[pl.ds(_LANES, _LANES)] = ftv1

        for r in range(_LANES):
            plsc.store_scatter(tbl, [lane * _LANES + r], ftv0)
            plsc.store_scatter(tbl, [(_LANES + lane) * _LANES + r], ftv1,
                               mask=lane < 8)

        xbufs, obufs = (xb0, xb1), (ob0, ob1)
        sins, souts = (si0, si1), (so0, so1)
        in_d, out_d = {}, {}

        def start_in(g):
            d = pltpu.make_async_copy(
                x_hbm.at[pl.ds(base + g * cb, cb)], xbufs[g % 2], sins[g % 2])
            d.start()
            in_d[g] = d

        def start_out(g):
            ds = []
            for c in range(3):
                d = pltpu.make_async_copy(
                    obufs[g % 2].at[pl.ds(c * cb, cb)],
                    out_hbm.at[pl.ds(c * n + base + g * cb, cb)],
                    souts[g % 2])
                d.start()
                ds.append(d)
            out_d[g] = ds

        def compute(g):
            xb, ob = xbufs[g % 2], obufs[g % 2]

            @plsc.parallel_loop(0, cb // _LANES, 1, unroll=8)
            def _(i):
                xv = xb[pl.ds(i * _LANES, _LANES)]
                idx = xv * (3 * _LANES) + lane
                for c in range(3):
                    ob[pl.ds(c * cb + i * _LANES, _LANES)] = (
                        plsc.load_gather(tbl, [idx + c * _LANES]))

        start_in(0)
        for g in range(ng):
            if g + 1 < ng:
                start_in(g + 1)
            in_d[g].wait()
            if g >= 2:
                for d in out_d[g - 2]:
                    d.wait()
            compute(g)
            start_out(g)
        for g in (ng - 2, ng - 1):
            for d in out_d[g]:
                d.wait()

    return emb_linear_sc


def kernel(x, emb, W, b):
    bsz, seq = x.shape
    n = bsz * seq
    it, im = bsz // 128, 128
    jt, js = seq // 8, 8
    xp = (x.reshape(it, im, jt, js)
           .transpose(2, 0, 3, 1)
           .reshape(-1))
    params = jnp.concatenate(
        [emb.reshape(-1), W.reshape(-1), b.reshape(-1),
         jnp.zeros((1,), jnp.float32)])
    op = _build(n, 12800)(xp, params)
    out = (op.reshape(3, jt, it, js, im)
             .transpose(2, 4, 1, 3, 0)
             .reshape(bsz, seq, 3))
    return out


# speedup vs baseline: 298.5401x; 1.0166x over previous
import functools

import jax
import jax.numpy as jnp
from jax import lax
from jax.experimental import pallas as pl
from jax.experimental.pallas import tpu as pltpu
from jax.experimental.pallas import tpu_sc as plsc

_NC = 2
_NS = 16
_NW = _NC * _NS
_LANES = 16


@functools.lru_cache(maxsize=None)
def _build(n: int, cb: int):
    assert n % (_NW * cb) == 0
    pw = n // _NW
    ng = pw // cb
    mesh = plsc.VectorSubcoreMesh(core_axis_name="c", subcore_axis_name="s",
                                  num_cores=_NC, num_subcores=_NS)

    @functools.partial(
        pl.kernel,
        out_type=jax.ShapeDtypeStruct((3 * n,), jnp.float32),
        mesh=mesh,
        scratch_types=[
            pltpu.VMEM((48,), jnp.float32),
            pltpu.VMEM((32,), jnp.float32),
            pltpu.VMEM((384,), jnp.float32),
            pltpu.VMEM((cb,), jnp.int32),
            pltpu.VMEM((cb,), jnp.int32),
            pltpu.VMEM((3 * cb,), jnp.float32),
            pltpu.VMEM((3 * cb,), jnp.float32),
            pltpu.SemaphoreType.DMA,
            pltpu.SemaphoreType.DMA,
            pltpu.SemaphoreType.DMA,
            pltpu.SemaphoreType.DMA,
        ],
        compiler_params=pltpu.CompilerParams(
            needs_layout_passes=False,
            disable_bounds_checks=True,
            disable_semaphore_checks=True),
    )
    def emb_linear_sc(x_hbm, p_hbm, out_hbm, p_v, ft_v, tbl,
                      xb0, xb1, ob0, ob1, si0, si1, so0, so1):
        wid = lax.axis_index("s") * _NC + lax.axis_index("c")
        base = wid * pw
        lane = jnp.arange(_LANES, dtype=jnp.int32)

        xbufs, obufs = (xb0, xb1), (ob0, ob1)
        sins, souts = (si0, si1), (so0, so1)
        in_d, out_d = {}, {}

        def start_in(g):
            d = pltpu.make_async_copy(
                x_hbm.at[pl.ds(base + g * cb, cb)], xbufs[g % 2], sins[g % 2])
            d.start()
            in_d[g] = d

        start_in(0)
        start_in(1)

        pltpu.sync_copy(p_hbm, p_v)

        def ft_half(e0):
            e = lane + e0
            k = jnp.minimum(e // 3, 7)
            c = e % 3
            acc = plsc.load_gather(p_v, [44 + c])
            for d in range(4):
                acc = acc + (plsc.load_gather(p_v, [k * 4 + d])
                             * plsc.load_gather(p_v, [32 + c * 4 + d]))
            return acc

        ftv0 = ft_half(0)
        ftv1 = ft_half(_LANES)
        ft_v[pl.ds(0, _LANES)] = ftv0
        ft_v[pl.ds(_LANES, _LANES)] = ftv1

        for r in range(_LANES):
            plsc.store_scatter(tbl, [lane * _LANES + r], ftv0)
            plsc.store_scatter(tbl, [(_LANES + lane) * _LANES + r], ftv1,
                               mask=lane < 8)

        def start_out(g):
            ds = []
            for c in range(3):
                d = pltpu.make_async_copy(
                    obufs[g % 2].at[pl.ds(c * cb, cb)],
                    out_hbm.at[pl.ds(c * n + base + g * cb, cb)],
                    souts[g % 2])
                d.start()
                ds.append(d)
            out_d[g] = ds

        def compute(g):
            xb, ob = xbufs[g % 2], obufs[g % 2]

            @plsc.parallel_loop(0, cb // _LANES, 1, unroll=8)
            def _(i):
                xv = xb[pl.ds(i * _LANES, _LANES)]
                idx = xv * (3 * _LANES) + lane
                for c in range(3):
                    ob[pl.ds(c * cb + i * _LANES, _LANES)] = (
                        plsc.load_gather(tbl, [idx + c * _LANES]))

        for g in range(ng):
            in_d[g].wait()
            if g >= 2:
                for d in out_d[g - 2]:
                    d.wait()
            compute(g)
            if g + 2 < ng:
                start_in(g + 2)
            start_out(g)
        for g in (ng - 2, ng - 1):
            for d in out_d[g]:
                d.wait()

    return emb_linear_sc


def kernel(x, emb, W, b):
    bsz, seq = x.shape
    n = bsz * seq
    it, im = bsz // 128, 128
    jt, js = seq // 8, 8
    xp = (x.reshape(it, im, jt, js)
           .transpose(2, 0, 3, 1)
           .reshape(-1))
    params = jnp.concatenate(
        [emb.reshape(-1), W.reshape(-1), b.reshape(-1),
         jnp.zeros((1,), jnp.float32)])
    op = _build(n, 12800)(xp, params)
    out = (op.reshape(3, jt, it, js, im)
             .transpose(2, 4, 1, 3, 0)
             .reshape(bsz, seq, 3))
    return out


# speedup vs baseline: 298.6181x; 1.0003x over previous
import functools

import jax
import jax.numpy as jnp
from jax import lax
from jax.experimental import pallas as pl
from jax.experimental.pallas import tpu as pltpu
from jax.experimental.pallas import tpu_sc as plsc

_NC = 2
_NS = 16
_NW = _NC * _NS
_LANES = 16


@functools.lru_cache(maxsize=None)
def _build(n: int, cb: int):
    assert n % (_NW * cb) == 0
    pw = n // _NW
    ng = pw // cb
    mesh = plsc.VectorSubcoreMesh(core_axis_name="c", subcore_axis_name="s",
                                  num_cores=_NC, num_subcores=_NS)

    @functools.partial(
        pl.kernel,
        out_type=jax.ShapeDtypeStruct((3 * n,), jnp.float32),
        mesh=mesh,
        scratch_types=[
            pltpu.VMEM((48,), jnp.float32),
            pltpu.VMEM((32,), jnp.float32),
            pltpu.VMEM((384,), jnp.float32),
            pltpu.VMEM((cb,), jnp.int32),
            pltpu.VMEM((cb,), jnp.int32),
            pltpu.VMEM((3 * cb,), jnp.float32),
            pltpu.VMEM((3 * cb,), jnp.float32),
            pltpu.SemaphoreType.DMA,
            pltpu.SemaphoreType.DMA,
            pltpu.SemaphoreType.DMA,
            pltpu.SemaphoreType.DMA,
        ],
        compiler_params=pltpu.CompilerParams(
            needs_layout_passes=False,
            disable_bounds_checks=True,
            disable_semaphore_checks=True,
            skip_device_barrier=True),
    )
    def emb_linear_sc(x_hbm, p_hbm, out_hbm, p_v, ft_v, tbl,
                      xb0, xb1, ob0, ob1, si0, si1, so0, so1):
        wid = lax.axis_index("s") * _NC + lax.axis_index("c")
        base = wid * pw
        lane = jnp.arange(_LANES, dtype=jnp.int32)

        xbufs, obufs = (xb0, xb1), (ob0, ob1)
        sins, souts = (si0, si1), (so0, so1)
        in_d, out_d = {}, {}

        def start_in(g):
            d = pltpu.make_async_copy(
                x_hbm.at[pl.ds(base + g * cb, cb)], xbufs[g % 2], sins[g % 2])
            d.start()
            in_d[g] = d

        start_in(0)
        start_in(1)

        pltpu.sync_copy(p_hbm, p_v)

        def ft_half(e0):
            e = lane + e0
            k = jnp.minimum(e // 3, 7)
            c = e % 3
            acc = plsc.load_gather(p_v, [44 + c])
            for d in range(4):
                acc = acc + (plsc.load_gather(p_v, [k * 4 + d])
                             * plsc.load_gather(p_v, [32 + c * 4 + d]))
            return acc

        ftv0 = ft_half(0)
        ftv1 = ft_half(_LANES)
        ft_v[pl.ds(0, _LANES)] = ftv0
        ft_v[pl.ds(_LANES, _LANES)] = ftv1

        for r in range(_LANES):
            plsc.store_scatter(tbl, [lane * _LANES + r], ftv0)
            plsc.store_scatter(tbl, [(_LANES + lane) * _LANES + r], ftv1,
                               mask=lane < 8)

        def start_out(g):
            ds = []
            for c in range(3):
                d = pltpu.make_async_copy(
                    obufs[g % 2].at[pl.ds(c * cb, cb)],
                    out_hbm.at[pl.ds(c * n + base + g * cb, cb)],
                    souts[g % 2])
                d.start()
                ds.append(d)
            out_d[g] = ds

        def compute(g):
            xb, ob = xbufs[g % 2], obufs[g % 2]

            @plsc.parallel_loop(0, cb // _LANES, 1, unroll=8)
            def _(i):
                xv = xb[pl.ds(i * _LANES, _LANES)]
                idx = xv * (3 * _LANES) + lane
                for c in range(3):
                    ob[pl.ds(c * cb + i * _LANES, _LANES)] = (
                        plsc.load_gather(tbl, [idx + c * _LANES]))

        for g in range(ng):
            in_d[g].wait()
            if g >= 2:
                for d in out_d[g - 2]:
                    d.wait()
            compute(g)
            if g + 2 < ng:
                start_in(g + 2)
            start_out(g)
        for g in (ng - 2, ng - 1):
            for d in out_d[g]:
                d.wait()

    return emb_linear_sc


def kernel(x, emb, W, b):
    bsz, seq = x.shape
    n = bsz * seq
    it, im = bsz // 128, 128
    jt, js = seq // 8, 8
    xp = (x.reshape(it, im, jt, js)
           .transpose(2, 0, 3, 1)
           .reshape(-1))
    params = jnp.concatenate(
        [emb.reshape(-1), W.reshape(-1), b.reshape(-1),
         jnp.zeros((1,), jnp.float32)])
    op = _build(n, 12800)(xp, params)
    out = (op.reshape(3, jt, it, js, im)
             .transpose(2, 4, 1, 3, 0)
             .reshape(bsz, seq, 3))
    return out
